# Initial kernel scaffold; baseline (speedup 1.0000x reference)
import jax
import jax.numpy as jnp
from jax.experimental import pallas as pl


def kernel(source, target, source_pred, target_pred, rho, rho_list):
    raise NotImplementedError("write your pallas kernel here")


# speedup vs baseline: 456.0128x; 456.0128x over previous
import functools

import jax
import jax.numpy as jnp
from jax.experimental import pallas as pl

_NS, _NT, _D, _DP = 512, 2048, 256, 128
_BS = 128


def _body(src, tgt, sp, tp, out1, out2, beta):
    s = src[:]
    t = tgt[:]
    a = sp[:]
    b = tp[:]

    dn = (((1,), (1,)), ((), ()))
    d2 = (jnp.sum(s * s, axis=1, keepdims=True)
          + jnp.sum(t * t, axis=1)[None, :]
          - 2.0 * jax.lax.dot_general(s, t, dn, preferred_element_type=jnp.float32))
    dist = jnp.sqrt(jnp.maximum(d2, 1e-12))
    l2 = (jnp.sum(a * a, axis=1, keepdims=True)
          + jnp.sum(b * b, axis=1)[None, :]
          - 2.0 * jax.lax.dot_general(a, b, dn, preferred_element_type=jnp.float32))
    sem = jnp.sqrt(jnp.maximum(l2, 1e-12))

    avg_d = jnp.mean(dist, axis=1, keepdims=True)
    avg_l = jnp.mean(sem, axis=1, keepdims=True)
    ratio = (dist - avg_d) / ((sem - avg_l) + 0.0001)
    p = ratio > 0.0

    inf = jnp.float32(jnp.inf)
    col = jax.lax.broadcasted_iota(jnp.int32, (_BS, _NT), 1)
    dmin = jnp.min(jnp.where(p, dist, inf), axis=1, keepdims=True)
    jmin = jnp.min(jnp.where(p & (dist == dmin), col, _NT), axis=1, keepdims=True)
    dmax = jnp.max(dist, axis=1, keepdims=True)
    jmax = jnp.max(jnp.where(dist == dmax, col, -1), axis=1, keepdims=True)
    has_pos = dmin < inf
    sel_d = jnp.where(has_pos, dmin, dmax)
    sel_j = jnp.where(has_pos, jmin, jmax)

    pos_mask = ((dist < sel_d) | ((dist == sel_d) & (col <= sel_j))
                ).astype(jnp.float32)
    kp1 = jnp.sum(pos_mask, axis=1, keepdims=True)
    pos_w = pos_mask / kp1
    neg_w = (1.0 - pos_mask) / jnp.maximum(jnp.float32(_NT) - kp1, 1.0)

    dn2 = (((1,), (0,)), ((), ()))
    out1[:] = jax.lax.dot_general(pos_w, t, dn2, preferred_element_type=jnp.float32)
    out2[:] = jax.lax.dot_general(neg_w, t, dn2, preferred_element_type=jnp.float32)
    onsel = (dist == sel_d) & (col == sel_j)
    beta[:] = jnp.sum(jnp.where(onsel, ratio, 0.0), axis=1, keepdims=True)


@functools.partial(jax.jit, static_argnames=())
def _run(source, target, source_pred, target_pred):
    grid = (_NS // _BS,)
    out1, out2, beta = pl.pallas_call(
        _body,
        grid=grid,
        in_specs=[
            pl.BlockSpec((_BS, _D), lambda i: (i, 0)),
            pl.BlockSpec((_NT, _D), lambda i: (0, 0)),
            pl.BlockSpec((_BS, _DP), lambda i: (i, 0)),
            pl.BlockSpec((_NT, _DP), lambda i: (0, 0)),
        ],
        out_specs=[
            pl.BlockSpec((_BS, _D), lambda i: (i, 0)),
            pl.BlockSpec((_BS, _D), lambda i: (i, 0)),
            pl.BlockSpec((_BS, 1), lambda i: (i, 0)),
        ],
        out_shape=[
            jax.ShapeDtypeStruct((_NS, _D), jnp.float32),
            jax.ShapeDtypeStruct((_NS, _D), jnp.float32),
            jax.ShapeDtypeStruct((_NS, 1), jnp.float32),
        ],
    )(source, target, source_pred, target_pred)
    return out1, out2, beta[:, 0]


def kernel(source, target, source_pred, target_pred, rho, rho_list):
    return _run(source, target, source_pred, target_pred)


# speedup vs baseline: 505.2711x; 1.1080x over previous
import functools

import jax
import jax.numpy as jnp
from jax.experimental import pallas as pl

_NS, _NT, _D, _DP = 512, 2048, 256, 128
_BS = 256


def _body(src, tgt, sp, tp, out1, out2, beta):
    s = src[:]
    t = tgt[:]
    a = sp[:]
    b = tp[:]

    dn = (((1,), (1,)), ((), ()))
    d2 = (jnp.sum(s * s, axis=1, keepdims=True)
          + jnp.sum(t * t, axis=1)[None, :]
          - 2.0 * jax.lax.dot_general(s, t, dn, preferred_element_type=jnp.float32))
    dist = jnp.sqrt(jnp.maximum(d2, 1e-12))
    l2 = (jnp.sum(a * a, axis=1, keepdims=True)
          + jnp.sum(b * b, axis=1)[None, :]
          - 2.0 * jax.lax.dot_general(a, b, dn, preferred_element_type=jnp.float32))
    sem = jnp.sqrt(jnp.maximum(l2, 1e-12))

    avg_d = jnp.mean(dist, axis=1, keepdims=True)
    avg_l = jnp.mean(sem, axis=1, keepdims=True)
    ratio = (dist - avg_d) / ((sem - avg_l) + 0.0001)
    p = ratio > 0.0

    inf = jnp.float32(jnp.inf)
    col = jax.lax.broadcasted_iota(jnp.int32, (_BS, _NT), 1)
    dmin = jnp.min(jnp.where(p, dist, inf), axis=1, keepdims=True)
    jmin = jnp.min(jnp.where(p & (dist == dmin), col, _NT), axis=1, keepdims=True)
    dmax = jnp.max(dist, axis=1, keepdims=True)
    jmax = jnp.max(jnp.where(dist == dmax, col, -1), axis=1, keepdims=True)
    has_pos = dmin < inf
    sel_d = jnp.where(has_pos, dmin, dmax)
    sel_j = jnp.where(has_pos, jmin, jmax)

    pos_mask = ((dist < sel_d) | ((dist == sel_d) & (col <= sel_j))
                ).astype(jnp.float32)
    kp1 = jnp.sum(pos_mask, axis=1, keepdims=True)

    dn2 = (((1,), (0,)), ((), ()))
    pos_sum = jax.lax.dot_general(pos_mask, t, dn2, preferred_element_type=jnp.float32)
    total = jnp.sum(t, axis=0, keepdims=True)
    negc = jnp.maximum(jnp.float32(_NT) - kp1, 1.0)
    out1[:] = pos_sum / kp1
    out2[:] = jnp.where(kp1 > _NT - 0.5, 0.0, (total - pos_sum) / negc)
    onsel = (dist == sel_d) & (col == sel_j)
    beta[:] = jnp.sum(jnp.where(onsel, ratio, 0.0), axis=1, keepdims=True)


@functools.partial(jax.jit, static_argnames=())
def _run(source, target, source_pred, target_pred):
    grid = (_NS // _BS,)
    out1, out2, beta = pl.pallas_call(
        _body,
        grid=grid,
        in_specs=[
            pl.BlockSpec((_BS, _D), lambda i: (i, 0)),
            pl.BlockSpec((_NT, _D), lambda i: (0, 0)),
            pl.BlockSpec((_BS, _DP), lambda i: (i, 0)),
            pl.BlockSpec((_NT, _DP), lambda i: (0, 0)),
        ],
        out_specs=[
            pl.BlockSpec((_BS, _D), lambda i: (i, 0)),
            pl.BlockSpec((_BS, _D), lambda i: (i, 0)),
            pl.BlockSpec((_BS, 1), lambda i: (i, 0)),
        ],
        out_shape=[
            jax.ShapeDtypeStruct((_NS, _D), jnp.float32),
            jax.ShapeDtypeStruct((_NS, _D), jnp.float32),
            jax.ShapeDtypeStruct((_NS, 1), jnp.float32),
        ],
    )(source, target, source_pred, target_pred)
    return out1, out2, beta[:, 0]


def kernel(source, target, source_pred, target_pred, rho, rho_list):
    return _run(source, target, source_pred, target_pred)


# speedup vs baseline: 532.5592x; 1.0540x over previous
import functools

import jax
import jax.numpy as jnp
from jax.experimental import pallas as pl

_NS, _NT, _D, _DP = 512, 2048, 256, 128
_BS = 512


def _body(src, tgt, sp, tp, out1, out2, beta):
    s = src[:]
    t = tgt[:]
    a = sp[:]
    b = tp[:]

    dn = (((1,), (1,)), ((), ()))
    d2 = (jnp.sum(s * s, axis=1, keepdims=True)
          + jnp.sum(t * t, axis=1)[None, :]
          - 2.0 * jax.lax.dot_general(s, t, dn, preferred_element_type=jnp.float32))
    dist = jnp.sqrt(jnp.maximum(d2, 1e-12))
    l2 = (jnp.sum(a * a, axis=1, keepdims=True)
          + jnp.sum(b * b, axis=1)[None, :]
          - 2.0 * jax.lax.dot_general(a, b, dn, preferred_element_type=jnp.float32))
    sem = jnp.sqrt(jnp.maximum(l2, 1e-12))

    avg_d = jnp.mean(dist, axis=1, keepdims=True)
    avg_l = jnp.mean(sem, axis=1, keepdims=True)
    ratio = (dist - avg_d) / ((sem - avg_l) + 0.0001)
    p = ratio > 0.0

    inf = jnp.float32(jnp.inf)
    col = jax.lax.broadcasted_iota(jnp.int32, (_BS, _NT), 1)
    dmin = jnp.min(jnp.where(p, dist, inf), axis=1, keepdims=True)
    jmin = jnp.min(jnp.where(p & (dist == dmin), col, _NT), axis=1, keepdims=True)
    dmax = jnp.max(dist, axis=1, keepdims=True)
    jmax = jnp.max(jnp.where(dist == dmax, col, -1), axis=1, keepdims=True)
    has_pos = dmin < inf
    sel_d = jnp.where(has_pos, dmin, dmax)
    sel_j = jnp.where(has_pos, jmin, jmax)

    pos_mask = ((dist < sel_d) | ((dist == sel_d) & (col <= sel_j))
                ).astype(jnp.float32)
    kp1 = jnp.sum(pos_mask, axis=1, keepdims=True)

    dn2 = (((1,), (0,)), ((), ()))
    pos_sum = jax.lax.dot_general(pos_mask, t, dn2, preferred_element_type=jnp.float32)
    total = jnp.sum(t, axis=0, keepdims=True)
    negc = jnp.maximum(jnp.float32(_NT) - kp1, 1.0)
    out1[:] = pos_sum / kp1
    out2[:] = jnp.where(kp1 > _NT - 0.5, 0.0, (total - pos_sum) / negc)
    onsel = (dist == sel_d) & (col == sel_j)
    beta[:] = jnp.sum(jnp.where(onsel, ratio, 0.0), axis=1, keepdims=True)


@functools.partial(jax.jit, static_argnames=())
def _run(source, target, source_pred, target_pred):
    grid = (_NS // _BS,)
    out1, out2, beta = pl.pallas_call(
        _body,
        grid=grid,
        in_specs=[
            pl.BlockSpec((_BS, _D), lambda i: (i, 0)),
            pl.BlockSpec((_NT, _D), lambda i: (0, 0)),
            pl.BlockSpec((_BS, _DP), lambda i: (i, 0)),
            pl.BlockSpec((_NT, _DP), lambda i: (0, 0)),
        ],
        out_specs=[
            pl.BlockSpec((_BS, _D), lambda i: (i, 0)),
            pl.BlockSpec((_BS, _D), lambda i: (i, 0)),
            pl.BlockSpec((_BS, 1), lambda i: (i, 0)),
        ],
        out_shape=[
            jax.ShapeDtypeStruct((_NS, _D), jnp.float32),
            jax.ShapeDtypeStruct((_NS, _D), jnp.float32),
            jax.ShapeDtypeStruct((_NS, 1), jnp.float32),
        ],
    )(source, target, source_pred, target_pred)
    return out1, out2, beta[:, 0]


def kernel(source, target, source_pred, target_pred, rho, rho_list):
    return _run(source, target, source_pred, target_pred)
